# baseline (speedup 1.0000x reference)
import functools

import jax
import jax.numpy as jnp
from jax import lax
from jax.experimental import pallas as pl
from jax.experimental.pallas import tpu as pltpu
from jax.experimental.pallas import tpu_sc as plsc

_D_PAD = 128
_LANES = 16
_CHUNK = 128
_ROW_BLOCK = 1000


def _matmul_body(t_ref, w_ref, b_ref, o_ref):
    o_ref[...] = (
        jnp.dot(t_ref[...], w_ref[...], preferred_element_type=jnp.float32)
        + b_ref[...]
    ).astype(o_ref.dtype)


def _project_table(table, w_pad, b_pad):
    v, k = table.shape
    d = w_pad.shape[1]
    return pl.pallas_call(
        _matmul_body,
        grid=(v // _ROW_BLOCK,),
        in_specs=[
            pl.BlockSpec((_ROW_BLOCK, k), lambda i: (i, 0)),
            pl.BlockSpec((k, d), lambda i: (0, 0)),
            pl.BlockSpec((1, d), lambda i: (0, 0)),
        ],
        out_specs=pl.BlockSpec((_ROW_BLOCK, d), lambda i: (i, 0)),
        out_shape=jax.ShapeDtypeStruct((v, d), jnp.float32),
    )(table, w_pad, b_pad)


@functools.lru_cache(maxsize=None)
def _make_gather(n_workers, d, bsz, seq, seq_pad):
    mesh = plsc.VectorSubcoreMesh(core_axis_name="c", subcore_axis_name="s")
    num_cores = mesh.num_cores
    b_per_w = bsz // n_workers

    @functools.partial(
        pl.kernel,
        out_type=jax.ShapeDtypeStruct((bsz, seq_pad, d), jnp.float32),
        mesh=mesh,
        scratch_types=[
            pltpu.VMEM((b_per_w, seq_pad), jnp.int32),
            pltpu.VMEM((b_per_w, seq_pad, d), jnp.float32),
            pltpu.SemaphoreType.DMA,
        ],
    )
    def gather(p_hbm, idx_hbm, out_hbm, idx_v, rows_v, sem):
        wid = lax.axis_index("s") * num_cores + lax.axis_index("c")
        pltpu.sync_copy(idx_hbm.at[wid], idx_v)
        copies = [
            pltpu.async_copy(p_hbm.at[idx_v.at[bb]], rows_v.at[bb], sem)
            for bb in range(b_per_w)
        ]
        for c in copies:
            c.wait()
        pltpu.sync_copy(rows_v, out_hbm.at[pl.ds(wid * b_per_w, b_per_w)])

    return gather


def kernel(input_ids, table, W, b):
    bsz, seq = input_ids.shape
    k, out_d = W.shape

    w_pad = jnp.zeros((k, _D_PAD), W.dtype).at[:, :out_d].set(W)
    b_pad = jnp.zeros((1, _D_PAD), b.dtype).at[0, :out_d].set(b)
    p = _project_table(table, w_pad, b_pad)

    n_workers = 32
    seq_pad = (seq + 7) // 8 * 8
    idx = jnp.pad(input_ids.astype(jnp.int32), ((0, 0), (0, seq_pad - seq)))
    idx = idx.reshape(n_workers, bsz // n_workers, seq_pad)

    rows = _make_gather(n_workers, _D_PAD, bsz, seq, seq_pad)(p, idx)
    return rows[:, :seq, :out_d]


# speedup vs baseline: 2.6830x; 2.6830x over previous
import functools

import jax
import jax.numpy as jnp
from jax import lax
from jax.experimental import pallas as pl
from jax.experimental.pallas import tpu as pltpu
from jax.experimental.pallas import tpu_sc as plsc

_D_PAD = 128
_LANES = 16
_CHUNK = 128
_ROW_BLOCK = 1000


def _matmul_body(t_ref, w_ref, b_ref, o_ref):
    o_ref[...] = (
        jnp.dot(t_ref[...], w_ref[...], preferred_element_type=jnp.float32)
        + b_ref[...]
    ).astype(o_ref.dtype)


def _project_table(table, w_pad, b_pad):
    v, k = table.shape
    d = w_pad.shape[1]
    return pl.pallas_call(
        _matmul_body,
        grid=(v // _ROW_BLOCK,),
        in_specs=[
            pl.BlockSpec((_ROW_BLOCK, k), lambda i: (i, 0)),
            pl.BlockSpec((k, d), lambda i: (0, 0)),
            pl.BlockSpec((1, d), lambda i: (0, 0)),
        ],
        out_specs=pl.BlockSpec((_ROW_BLOCK, d), lambda i: (i, 0)),
        out_shape=jax.ShapeDtypeStruct((v, d), jnp.float32),
    )(table, w_pad, b_pad)


@functools.lru_cache(maxsize=None)
def _make_gather(n_workers, d, bsz, seq, seq_pad):
    mesh = plsc.VectorSubcoreMesh(core_axis_name="c", subcore_axis_name="s")
    num_cores = mesh.num_cores
    b_per_w = bsz // n_workers
    n_rows = b_per_w * seq
    n_chunks = n_rows // _CHUNK

    @functools.partial(
        pl.kernel,
        out_type=jax.ShapeDtypeStruct((bsz, seq_pad, d), jnp.float32),
        mesh=mesh,
        scratch_types=[
            pltpu.VMEM((n_chunks, _CHUNK), jnp.int32),
            pltpu.VMEM((n_rows + seq_pad - seq, d), jnp.float32),
            pltpu.SemaphoreType.DMA,
            pltpu.SemaphoreType.DMA,
        ],
    )
    def gather(p_hbm, idx_hbm, out_hbm, idx_v, rows_v, sem, out_sem):
        wid = lax.axis_index("s") * num_cores + lax.axis_index("c")
        pltpu.sync_copy(idx_hbm.at[wid], idx_v)
        copies = [
            pltpu.async_copy(p_hbm.at[idx_v.at[j]],
                             rows_v.at[pl.ds(j * _CHUNK, _CHUNK)], sem)
            for j in range(n_chunks)
        ]
        for c in copies:
            c.wait()
        b0 = wid * b_per_w
        writes = [
            pltpu.async_copy(rows_v.at[pl.ds(bb * seq, seq_pad)],
                             out_hbm.at[b0 + bb], out_sem)
            for bb in range(b_per_w)
        ]
        for wr in writes:
            wr.wait()

    return gather


def kernel(input_ids, table, W, b):
    bsz, seq = input_ids.shape
    k, out_d = W.shape

    w_pad = jnp.zeros((k, _D_PAD), W.dtype).at[:, :out_d].set(W)
    b_pad = jnp.zeros((1, _D_PAD), b.dtype).at[0, :out_d].set(b)
    p = _project_table(table, w_pad, b_pad)

    n_workers = 32
    seq_pad = (seq + 7) // 8 * 8
    n = bsz * seq
    idx = input_ids.astype(jnp.int32).reshape(
        n_workers, n // (n_workers * _CHUNK), _CHUNK)

    rows = _make_gather(n_workers, _D_PAD, bsz, seq, seq_pad)(p, idx)
    return rows[:, :seq, :out_d]


# speedup vs baseline: 2.6956x; 1.0047x over previous
import functools

import jax
import jax.numpy as jnp
from jax import lax
from jax.experimental import pallas as pl
from jax.experimental.pallas import tpu as pltpu
from jax.experimental.pallas import tpu_sc as plsc

_D_PAD = 128
_LANES = 16
_CHUNK = 128
_ROW_BLOCK = 1000


def _matmul_body(t_ref, w_ref, b_ref, o_ref):
    o_ref[...] = (
        jnp.dot(t_ref[...], w_ref[...], preferred_element_type=jnp.float32)
        + b_ref[...]
    ).astype(o_ref.dtype)


def _project_table(table, w_pad, b_pad):
    v, k = table.shape
    d = w_pad.shape[1]
    return pl.pallas_call(
        _matmul_body,
        grid=(v // _ROW_BLOCK,),
        in_specs=[
            pl.BlockSpec((_ROW_BLOCK, k), lambda i: (i, 0)),
            pl.BlockSpec((k, d), lambda i: (0, 0)),
            pl.BlockSpec((1, d), lambda i: (0, 0)),
        ],
        out_specs=pl.BlockSpec((_ROW_BLOCK, d), lambda i: (i, 0)),
        out_shape=jax.ShapeDtypeStruct((v, d), jnp.float32),
    )(table, w_pad, b_pad)


@functools.lru_cache(maxsize=None)
def _make_gather(n_workers, d, bsz, seq, seq_pad):
    mesh = plsc.VectorSubcoreMesh(core_axis_name="c", subcore_axis_name="s")
    num_cores = mesh.num_cores
    b_per_w = bsz // n_workers
    n_rows = b_per_w * seq
    n_chunks = n_rows // _CHUNK

    ready_after = [
        min(j for j in range(n_chunks)
            if bb * seq + seq_pad <= (j + 1) * _CHUNK or j == n_chunks - 1)
        for bb in range(b_per_w)
    ]

    @functools.partial(
        pl.kernel,
        out_type=jax.ShapeDtypeStruct((bsz, seq_pad, d), jnp.float32),
        mesh=mesh,
        scratch_types=[
            pltpu.VMEM((n_chunks, _CHUNK), jnp.int32),
            pltpu.VMEM((n_rows + seq_pad - seq, d), jnp.float32),
            [pltpu.SemaphoreType.DMA] * n_chunks,
            pltpu.SemaphoreType.DMA,
        ],
    )
    def gather(p_hbm, idx_hbm, out_hbm, idx_v, rows_v, sems, out_sem):
        wid = lax.axis_index("s") * num_cores + lax.axis_index("c")
        pltpu.sync_copy(idx_hbm.at[wid], idx_v)
        copies = [
            pltpu.async_copy(p_hbm.at[idx_v.at[j]],
                             rows_v.at[pl.ds(j * _CHUNK, _CHUNK)], sems[j])
            for j in range(n_chunks)
        ]
        b0 = wid * b_per_w
        writes = []
        for j in range(n_chunks):
            copies[j].wait()
            writes += [
                pltpu.async_copy(rows_v.at[pl.ds(bb * seq, seq_pad)],
                                 out_hbm.at[b0 + bb], out_sem)
                for bb in range(b_per_w)
                if ready_after[bb] == j
            ]
        for wr in writes:
            wr.wait()

    return gather


def kernel(input_ids, table, W, b):
    bsz, seq = input_ids.shape
    k, out_d = W.shape

    w_pad = jnp.zeros((k, _D_PAD), W.dtype).at[:, :out_d].set(W)
    b_pad = jnp.zeros((1, _D_PAD), b.dtype).at[0, :out_d].set(b)
    p = _project_table(table, w_pad, b_pad)

    n_workers = 32
    seq_pad = (seq + 7) // 8 * 8
    n = bsz * seq
    idx = input_ids.astype(jnp.int32).reshape(
        n_workers, n // (n_workers * _CHUNK), _CHUNK)

    rows = _make_gather(n_workers, _D_PAD, bsz, seq, seq_pad)(p, idx)
    return rows[:, :seq, :out_d]
